# baseline (device time: 144445 ns/iter reference)
import jax
import jax.numpy as jnp
from jax import lax
from jax.experimental import pallas as pl
from jax.experimental.pallas import tpu as pltpu

S = 1024
D = 2048
DC = 128
H = 16
DH = 128
DR = 32
DP = 256
SCALE = (DH + DR) ** -0.5

BF16 = jnp.bfloat16
F32 = jnp.float32


def _kv_comm_body(x_ref, wdkv_ref, wuk_ref, wuv_ref, wkr_ref,
                  kc_ref, v_ref,
                  s_wdkv, s_wuk, s_wuv, r_wdkv, r_wuk, r_wuv,
                  send_sems, recv_sems):
    my_x = lax.axis_index("x")
    my_y = lax.axis_index("y")
    partner = (1 - my_x, my_y)

    barrier = pltpu.get_barrier_semaphore()
    pl.semaphore_signal(barrier, inc=1, device_id=partner,
                        device_id_type=pl.DeviceIdType.MESH)
    pl.semaphore_wait(barrier, 1)

    s_wdkv[...] = wdkv_ref[...].astype(BF16)
    s_wuk[...] = wuk_ref[...].astype(BF16)
    s_wuv[...] = wuv_ref[...].astype(BF16)
    rdmas = []
    for i, (s, r) in enumerate([(s_wdkv, r_wdkv), (s_wuk, r_wuk),
                                (s_wuv, r_wuv)]):
        rdma = pltpu.make_async_remote_copy(
            src_ref=s, dst_ref=r,
            send_sem=send_sems.at[i], recv_sem=recv_sems.at[i],
            device_id=partner, device_id_type=pl.DeviceIdType.MESH,
        )
        rdma.start()
        rdmas.append(rdma)

    x16 = x_ref[0].astype(BF16)
    c_loc = jnp.dot(x16, s_wdkv[...], preferred_element_type=F32).astype(BF16)
    k_acc = jnp.dot(c_loc, s_wuk[...], preferred_element_type=F32)
    v_acc = jnp.dot(c_loc, s_wuv[...], preferred_element_type=F32)
    kr = jnp.dot(x16, wkr_ref[...].astype(BF16),
                 preferred_element_type=F32).astype(BF16)

    for rdma in rdmas:
        rdma.wait()

    c_rem = jnp.dot(x16, r_wdkv[...], preferred_element_type=F32).astype(BF16)
    k = (k_acc + jnp.dot(c_rem, r_wuk[...],
                         preferred_element_type=F32)).astype(BF16)
    v_ref[...] = (v_acc + jnp.dot(c_rem, r_wuv[...],
                                  preferred_element_type=F32)).astype(BF16)

    kc_ref[...] = jnp.zeros((S, H * DP), BF16)
    for h in range(H):
        kc_ref[:, h * DP:h * DP + DH] = k[:, h * DH:(h + 1) * DH]
        kc_ref[:, h * DP + DH:h * DP + DH + DR] = kr


def _q_proj_body(x_ref, wq_ref, wqr_ref, qc_ref):
    x16 = x_ref[0].astype(BF16)
    qc_ref[...] = jnp.zeros((S, H * DP), BF16)
    for i in range(H // 2):
        q2 = jnp.dot(x16, wq_ref[:, i * 256:(i + 1) * 256].astype(BF16),
                     preferred_element_type=F32).astype(BF16)
        h0, h1 = 2 * i, 2 * i + 1
        qc_ref[:, h0 * DP:h0 * DP + DH] = q2[:, :DH]
        qc_ref[:, h1 * DP:h1 * DP + DH] = q2[:, DH:]
    qr = jnp.dot(x16, wqr_ref[...].astype(BF16),
                 preferred_element_type=F32).astype(BF16)
    for h in range(H):
        qc_ref[:, h * DP + DH:h * DP + DH + DR] = qr[:, h * DR:(h + 1) * DR]


def _attn_body(qc_ref, kc_ref, v_ref, wo_ref, o_ref):
    h = pl.program_id(0)
    s = lax.dot_general(qc_ref[...], kc_ref[...],
                        (((1,), (1,)), ((), ())),
                        preferred_element_type=F32) * SCALE
    m = jnp.max(s, axis=1, keepdims=True)
    p = jnp.exp(s - m)
    p = (p / jnp.sum(p, axis=1, keepdims=True)).astype(BF16)
    o_h = jnp.dot(p, v_ref[...], preferred_element_type=F32).astype(BF16)
    contrib = jnp.dot(o_h, wo_ref[...].astype(BF16),
                      preferred_element_type=F32)

    @pl.when(h == 0)
    def _():
        o_ref[0] = contrib

    @pl.when(h != 0)
    def _():
        o_ref[0] = o_ref[0] + contrib


def kernel(x, Wdkv, Wuk, Wuv, Wq, Wqr, Wkr, Wo):
    kc, v = pl.pallas_call(
        _kv_comm_body,
        out_shape=[
            jax.ShapeDtypeStruct((S, H * DP), BF16),
            jax.ShapeDtypeStruct((S, D), BF16),
        ],
        in_specs=[pl.BlockSpec(memory_space=pltpu.VMEM)] * 5,
        out_specs=[pl.BlockSpec(memory_space=pltpu.VMEM)] * 2,
        scratch_shapes=[
            pltpu.VMEM((D, DC), BF16),
            pltpu.VMEM((DC, D), BF16),
            pltpu.VMEM((DC, D), BF16),
            pltpu.VMEM((D, DC), BF16),
            pltpu.VMEM((DC, D), BF16),
            pltpu.VMEM((DC, D), BF16),
            pltpu.SemaphoreType.DMA((3,)),
            pltpu.SemaphoreType.DMA((3,)),
        ],
        compiler_params=pltpu.CompilerParams(collective_id=0),
    )(x, Wdkv, Wuk, Wuv, Wkr)

    qc = pl.pallas_call(
        _q_proj_body,
        out_shape=jax.ShapeDtypeStruct((S, H * DP), BF16),
        in_specs=[pl.BlockSpec(memory_space=pltpu.VMEM)] * 3,
        out_specs=pl.BlockSpec(memory_space=pltpu.VMEM),
    )(x, Wq, Wqr)

    out = pl.pallas_call(
        _attn_body,
        grid=(H,),
        out_shape=jax.ShapeDtypeStruct((1, S, D), F32),
        in_specs=[
            pl.BlockSpec((S, DP), lambda h: (0, h)),
            pl.BlockSpec((S, DP), lambda h: (0, h)),
            pl.BlockSpec((S, DH), lambda h: (0, h)),
            pl.BlockSpec((DH, D), lambda h: (h, 0)),
        ],
        out_specs=pl.BlockSpec((1, S, D), lambda h: (0, 0, 0)),
    )(qc, kc, v, Wo)
    return out


# device time: 114711 ns/iter; 1.2592x vs baseline; 1.2592x over previous
import jax
import jax.numpy as jnp
from jax import lax
from jax.experimental import pallas as pl
from jax.experimental.pallas import tpu as pltpu

S = 1024
SQ = 256
D = 2048
DC = 128
H = 16
DH = 128
DR = 32
SCALE = (DH + DR) ** -0.5

BF16 = jnp.bfloat16
F32 = jnp.float32
MESH = pl.DeviceIdType.MESH


def _kv_comm_body(x_ref, wdkv_ref, wuk_ref, wuv_ref, wkrp_ref, wqrp_ref,
                  k_ref, v_ref, krp_ref, xq_ref, qrp_ref,
                  s_wdkv, s_wuk, s_wuv, r_wdkv, r_wuk, r_wuv,
                  send_sems, recv_sems):
    my_x = lax.axis_index("x")
    my_y = lax.axis_index("y")
    partner = (1 - my_x, my_y)
    qidx = my_x * 2 + my_y

    barrier = pltpu.get_barrier_semaphore()
    pl.semaphore_signal(barrier, inc=1, device_id=partner,
                        device_id_type=MESH)
    pl.semaphore_wait(barrier, 1)

    s_wdkv[...] = wdkv_ref[...].astype(BF16)
    s_wuk[...] = wuk_ref[...].astype(BF16)
    s_wuv[...] = wuv_ref[...].astype(BF16)
    rdmas = []
    for i, (s, r) in enumerate([(s_wdkv, r_wdkv), (s_wuk, r_wuk),
                                (s_wuv, r_wuv)]):
        rdma = pltpu.make_async_remote_copy(
            src_ref=s, dst_ref=r,
            send_sem=send_sems.at[i], recv_sem=recv_sems.at[i],
            device_id=partner, device_id_type=MESH,
        )
        rdma.start()
        rdmas.append(rdma)

    x16 = x_ref[0].astype(BF16)

    def _pick(i):
        @pl.when(qidx == i)
        def _():
            xq_ref[...] = x16[i * SQ:(i + 1) * SQ, :]
    for i in range(4):
        _pick(i)
    xq = xq_ref[...]
    c_loc = jnp.dot(x16, s_wdkv[...], preferred_element_type=F32).astype(BF16)
    k_acc = jnp.dot(c_loc, s_wuk[...], preferred_element_type=F32).astype(BF16)
    v_acc = jnp.dot(c_loc, s_wuv[...], preferred_element_type=F32).astype(BF16)
    krp_ref[...] = jnp.dot(x16, wkrp_ref[...],
                           preferred_element_type=F32).astype(BF16)
    qrp_ref[...] = jnp.dot(xq, wqrp_ref[...],
                           preferred_element_type=F32).astype(BF16)

    for rdma in rdmas:
        rdma.wait()

    c_rem = jnp.dot(x16, r_wdkv[...], preferred_element_type=F32).astype(BF16)
    k_ref[...] = (k_acc + jnp.dot(c_rem, r_wuk[...],
                                  preferred_element_type=F32)).astype(BF16)
    v_ref[...] = (v_acc + jnp.dot(c_rem, r_wuv[...],
                                  preferred_element_type=F32)).astype(BF16)


def _q_proj_body(xq_ref, wq_ref, qq_ref):
    qq_ref[...] = jnp.dot(xq_ref[...], wq_ref[...].astype(BF16),
                          preferred_element_type=F32).astype(BF16)


def _attn_body(qq_ref, qrp_ref, k_ref, krp_ref, v_ref, wo_ref, o_ref):
    h = pl.program_id(0)
    s = lax.dot_general(qq_ref[...], k_ref[...], (((1,), (1,)), ((), ())),
                        preferred_element_type=F32)
    s = s + lax.dot_general(qrp_ref[...], krp_ref[...],
                            (((1,), (1,)), ((), ())),
                            preferred_element_type=F32)
    s = s * SCALE
    m = jnp.max(s, axis=1, keepdims=True)
    p = jnp.exp(s - m)
    p = (p / jnp.sum(p, axis=1, keepdims=True)).astype(BF16)
    o_h = jnp.dot(p, v_ref[...], preferred_element_type=F32).astype(BF16)
    contrib = jnp.dot(o_h, wo_ref[...].astype(BF16),
                      preferred_element_type=F32)

    @pl.when(h == 0)
    def _():
        o_ref[...] = contrib

    @pl.when(h != 0)
    def _():
        o_ref[...] = o_ref[...] + contrib


def _gather_body(oq_ref, out_ref, sbuf, rx, ry, rd, send_sems, recv_sems):
    my_x = lax.axis_index("x")
    my_y = lax.axis_index("y")
    px = (1 - my_x, my_y)
    py = (my_x, 1 - my_y)
    pd = (1 - my_x, 1 - my_y)
    q_me = my_x * 2 + my_y
    q_x = (1 - my_x) * 2 + my_y
    q_y = my_x * 2 + (1 - my_y)
    q_d = (1 - my_x) * 2 + (1 - my_y)

    barrier = pltpu.get_barrier_semaphore()
    for p in (px, py, pd):
        pl.semaphore_signal(barrier, inc=1, device_id=p, device_id_type=MESH)
    pl.semaphore_wait(barrier, 3)

    sbuf[...] = oq_ref[...].astype(BF16)

    rdmas = []
    for i, (p, r) in enumerate([(px, rx), (py, ry), (pd, rd)]):
        rdma = pltpu.make_async_remote_copy(
            src_ref=sbuf, dst_ref=r,
            send_sem=send_sems.at[i], recv_sem=recv_sems.at[i],
            device_id=p, device_id_type=MESH,
        )
        rdma.start()
        rdmas.append(rdma)
    for rdma in rdmas:
        rdma.wait()

    def _store(qq):
        @pl.when(qq == q_me)
        def _():
            out_ref[0, qq * SQ:(qq + 1) * SQ, :] = oq_ref[...]
        for q, r in ((q_x, rx), (q_y, ry), (q_d, rd)):
            @pl.when(qq == q)
            def _(r=r):
                out_ref[0, qq * SQ:(qq + 1) * SQ, :] = r[...].astype(F32)
    for qq in range(4):
        _store(qq)


def kernel(x, Wdkv, Wuk, Wuv, Wq, Wqr, Wkr, Wo):
    wqrp = jnp.pad(Wqr.reshape(D, H, DR),
                   ((0, 0), (0, 0), (0, DH - DR))).reshape(D, H * DH)
    wqrp = wqrp.astype(BF16)
    wkrp = jnp.pad(Wkr, ((0, 0), (0, DH - DR))).astype(BF16)

    k, v, krp, xq, qrp = pl.pallas_call(
        _kv_comm_body,
        out_shape=[
            jax.ShapeDtypeStruct((S, D), BF16),
            jax.ShapeDtypeStruct((S, D), BF16),
            jax.ShapeDtypeStruct((S, DH), BF16),
            jax.ShapeDtypeStruct((SQ, D), BF16),
            jax.ShapeDtypeStruct((SQ, H * DH), BF16),
        ],
        in_specs=[pl.BlockSpec(memory_space=pltpu.VMEM)] * 6,
        out_specs=[pl.BlockSpec(memory_space=pltpu.VMEM)] * 5,
        scratch_shapes=[
            pltpu.VMEM((D, DC), BF16),
            pltpu.VMEM((DC, D), BF16),
            pltpu.VMEM((DC, D), BF16),
            pltpu.VMEM((D, DC), BF16),
            pltpu.VMEM((DC, D), BF16),
            pltpu.VMEM((DC, D), BF16),
            pltpu.SemaphoreType.DMA((3,)),
            pltpu.SemaphoreType.DMA((3,)),
        ],
        compiler_params=pltpu.CompilerParams(collective_id=0),
    )(x, Wdkv, Wuk, Wuv, wkrp, wqrp)

    qq = pl.pallas_call(
        _q_proj_body,
        grid=(8,),
        out_shape=jax.ShapeDtypeStruct((SQ, D), BF16),
        in_specs=[
            pl.BlockSpec((SQ, D), lambda i: (0, 0)),
            pl.BlockSpec((D, D // 8), lambda i: (0, i)),
        ],
        out_specs=pl.BlockSpec((SQ, D // 8), lambda i: (0, i)),
    )(xq, Wq)

    oq = pl.pallas_call(
        _attn_body,
        grid=(H,),
        out_shape=jax.ShapeDtypeStruct((SQ, D), F32),
        in_specs=[
            pl.BlockSpec((SQ, DH), lambda h: (0, h)),
            pl.BlockSpec((SQ, DH), lambda h: (0, h)),
            pl.BlockSpec((S, DH), lambda h: (0, h)),
            pl.BlockSpec((S, DH), lambda h: (0, 0)),
            pl.BlockSpec((S, DH), lambda h: (0, h)),
            pl.BlockSpec((DH, D), lambda h: (h, 0)),
        ],
        out_specs=pl.BlockSpec((SQ, D), lambda h: (0, 0)),
    )(qq, qrp, k, krp, v, Wo)

    out = pl.pallas_call(
        _gather_body,
        out_shape=jax.ShapeDtypeStruct((1, S, D), F32),
        in_specs=[pl.BlockSpec(memory_space=pltpu.VMEM)],
        out_specs=pl.BlockSpec(memory_space=pltpu.VMEM),
        scratch_shapes=[
            pltpu.VMEM((SQ, D), BF16),
            pltpu.VMEM((SQ, D), BF16),
            pltpu.VMEM((SQ, D), BF16),
            pltpu.VMEM((SQ, D), BF16),
            pltpu.SemaphoreType.DMA((3,)),
            pltpu.SemaphoreType.DMA((3,)),
        ],
        compiler_params=pltpu.CompilerParams(collective_id=1),
    )(oq)
    return out
